# Initial kernel scaffold; baseline (speedup 1.0000x reference)
import jax
import jax.numpy as jnp
from jax.experimental import pallas as pl


def kernel(pred_reg, pred_obj, pred_cls, gt_points, stride):
    raise NotImplementedError("write your pallas kernel here")


# speedup vs baseline: 15.8593x; 15.8593x over previous
import jax
import jax.numpy as jnp
from jax.experimental import pallas as pl
from jax.experimental.pallas import tpu as pltpu

_B = 8
_NG = 8
_H = 128
_W = 128
_ETA_PX = 3.0
_POS_W = 1.2
_LAMBDA_CD = 1.0
_K_POS_CAP = 96
_LAMBDA_P0 = 1.0
_FINF_BITS = 0x7F800000


def _log_sigmoid(x):
    return jnp.minimum(x, 0.0) - jnp.log1p(jnp.exp(-jnp.abs(x)))


def _seg_dist(px, py, x1, y1, x2, y2):
    vx, vy = x2 - x1, y2 - y1
    wx, wy = px - x1, py - y1
    vv = vx * vx + vy * vy + 1e-09
    t = jnp.clip((wx * vx + wy * vy) / vv, 0.0, 1.0)
    dx = px - (x1 + t * vx)
    dy = py - (y1 + t * vy)
    return jnp.sqrt(dx * dx + dy * dy + 1e-12)


def _loss_kernel(gt_ref, s_ref, reg_ref, obj_ref, cls_ref, out_ref,
                 acc_ref, objt_ref, mfsum_ref):
    b = pl.program_id(0)
    j = pl.program_id(1)
    s = s_ref[0]

    @pl.when((b == 0) & (j == 0))
    def _init():
        for i in range(5):
            acc_ref[i] = 0.0

    @pl.when(j == 0)
    def _init_planes():
        objt_ref[...] = jnp.zeros((_H, _W), jnp.float32)
        mfsum_ref[...] = jnp.zeros((_H, _W), jnp.float32)

    iy = jax.lax.broadcasted_iota(jnp.int32, (_H, _W), 0).astype(jnp.float32)
    ix = jax.lax.broadcasted_iota(jnp.int32, (_H, _W), 1).astype(jnp.float32)
    yy = (iy + 0.5) * s
    xx = (ix + 0.5) * s

    ax = gt_ref[b, j, 0, 0]
    ay = gt_ref[b, j, 0, 1]
    bx = gt_ref[b, j, 1, 0]
    by = gt_ref[b, j, 1, 1]
    cx = gt_ref[b, j, 2, 0]
    cy = gt_ref[b, j, 2, 1]

    def sign(x1, y1, x2, y2, x3, y3):
        return (x1 - x3) * (y2 - y3) - (x2 - x3) * (y1 - y3)

    d1 = sign(xx, yy, ax, ay, bx, by)
    d2 = sign(xx, yy, bx, by, cx, cy)
    d3 = sign(xx, yy, cx, cy, ax, ay)
    has_neg = (d1 < 0) | (d2 < 0) | (d3 < 0)
    has_pos = (d1 > 0) | (d2 > 0) | (d3 > 0)
    inside = ~(has_neg & has_pos)
    dist = jnp.minimum(_seg_dist(xx, yy, ax, ay, bx, by),
                       jnp.minimum(_seg_dist(xx, yy, bx, by, cx, cy),
                                   _seg_dist(xx, yy, cx, cy, ax, ay)))

    mask = inside | (dist <= _ETA_PX)
    npix = jnp.sum(mask.astype(jnp.int32))

    dist_bits = jax.lax.bitcast_convert_type(dist, jnp.int32)
    key = jnp.where(mask, dist_bits, jnp.int32(0x7FFFFFFF))

    def bs_body(_, carry):
        lo, hi = carry
        mid = lo + (hi - lo) // 2
        cnt = jnp.sum((key <= mid).astype(jnp.int32))
        take = cnt >= _K_POS_CAP
        return (jnp.where(take, lo, mid + 1), jnp.where(take, mid, hi))

    lo, _hi = jax.lax.fori_loop(
        0, 31, bs_body, (jnp.int32(0), jnp.int32(_FINF_BITS)))

    mask_f = mask & ((npix <= _K_POS_CAP) | (dist_bits <= lo))
    mf = mask_f.astype(jnp.float32)

    objt_ref[...] = jnp.maximum(objt_ref[...], mf)
    mfsum_ref[...] = mfsum_ref[...] + mf

    g0x = (ax - xx) / s
    g0y = (ay - yy) / s
    p0 = (reg_ref[0, 0] - g0x) ** 2 + (reg_ref[0, 1] - g0y) ** 2

    g1x = (bx - xx) / s
    g1y = (by - yy) / s
    g2x = (cx - xx) / s
    g2y = (cy - yy) / s
    p1x, p1y = reg_ref[0, 2], reg_ref[0, 3]
    p2x, p2y = reg_ref[0, 4], reg_ref[0, 5]

    def pdist(px_, py_, gx_, gy_):
        return jnp.sqrt((px_ - gx_) ** 2 + (py_ - gy_) ** 2 + 1e-12)

    d11 = pdist(p1x, p1y, g1x, g1y)
    d12 = pdist(p1x, p1y, g2x, g2y)
    d21 = pdist(p2x, p2y, g1x, g1y)
    d22 = pdist(p2x, p2y, g2x, g2y)
    cd = (jnp.minimum(d11, d12) + jnp.minimum(d21, d22)
          + jnp.minimum(d11, d21) + jnp.minimum(d12, d22))

    step_reg = jnp.sum(mf * (_LAMBDA_P0 * p0 + _LAMBDA_CD * cd))
    acc_ref[0] = acc_ref[0] + step_reg
    acc_ref[3] = acc_ref[3] + jnp.minimum(npix, _K_POS_CAP).astype(jnp.float32)

    @pl.when(j == _NG - 1)
    def _finish_image():
        obj_t = objt_ref[...]
        x = obj_ref[0, 0]
        obj_l = jnp.sum(-(_POS_W * obj_t * _log_sigmoid(x)
                          + (1.0 - obj_t) * _log_sigmoid(-x)))
        cls_l = jnp.sum(mfsum_ref[...] * (-_log_sigmoid(cls_ref[0, 0])))
        neg = jnp.float32(_H * _W) - jnp.sum((obj_t > 0.5).astype(jnp.float32))
        acc_ref[1] = acc_ref[1] + obj_l
        acc_ref[2] = acc_ref[2] + cls_l
        acc_ref[4] = acc_ref[4] + neg

        @pl.when(b == _B - 1)
        def _finalize():
            pos_eps = jnp.maximum(acc_ref[3], 1.0)
            neg_eps = jnp.maximum(acc_ref[4], 1.0)
            out_ref[0] = (acc_ref[0] / pos_eps
                          + acc_ref[1] / (pos_eps + neg_eps)
                          + acc_ref[2] / pos_eps)


def kernel(pred_reg, pred_obj, pred_cls, gt_points, stride):
    s = jnp.asarray(stride, jnp.float32).reshape(1)
    out = pl.pallas_call(
        _loss_kernel,
        grid=(_B, _NG),
        in_specs=[
            pl.BlockSpec(memory_space=pltpu.SMEM),
            pl.BlockSpec(memory_space=pltpu.SMEM),
            pl.BlockSpec((1, 6, _H, _W), lambda b, j: (b, 0, 0, 0)),
            pl.BlockSpec((1, 1, _H, _W), lambda b, j: (b, 0, 0, 0)),
            pl.BlockSpec((1, 1, _H, _W), lambda b, j: (b, 0, 0, 0)),
        ],
        out_specs=pl.BlockSpec(memory_space=pltpu.SMEM),
        out_shape=jax.ShapeDtypeStruct((1,), jnp.float32),
        scratch_shapes=[
            pltpu.SMEM((5,), jnp.float32),
            pltpu.VMEM((_H, _W), jnp.float32),
            pltpu.VMEM((_H, _W), jnp.float32),
        ],
    )(gt_points, s, pred_reg, pred_obj, pred_cls)
    return out[0]


# speedup vs baseline: 42.7798x; 2.6975x over previous
import jax
import jax.numpy as jnp
from jax.experimental import pallas as pl
from jax.experimental.pallas import tpu as pltpu

_B = 8
_NG = 8
_H = 128
_W = 128
_ETA_PX = 3.0
_POS_W = 1.2
_LAMBDA_CD = 1.0
_K_POS_CAP = 96
_LAMBDA_P0 = 1.0
_FINF_BITS = 0x7F800000
_MAXI = 0x7FFFFFFF


def _log_sigmoid(x):
    return jnp.minimum(x, 0.0) - jnp.log1p(jnp.exp(-jnp.abs(x)))


def _seg_dist(px, py, x1, y1, x2, y2):
    vx, vy = x2 - x1, y2 - y1
    wx, wy = px - x1, py - y1
    vv = vx * vx + vy * vy + 1e-09
    t = jnp.clip((wx * vx + wy * vy) / vv, 0.0, 1.0)
    dx = px - (x1 + t * vx)
    dy = py - (y1 + t * vy)
    return jnp.sqrt(dx * dx + dy * dy + 1e-12)


def _loss_kernel(gt_ref, s_ref, reg_ref, obj_ref, cls_ref, out_ref,
                 acc_ref, key_ref, loss_ref):
    b = pl.program_id(0)
    s = s_ref[0]

    @pl.when(b == 0)
    def _init():
        for i in range(5):
            acc_ref[i] = 0.0

    iy = jax.lax.broadcasted_iota(jnp.int32, (_H, _W), 0).astype(jnp.float32)
    ix = jax.lax.broadcasted_iota(jnp.int32, (_H, _W), 1).astype(jnp.float32)
    yy = (iy + 0.5) * s
    xx = (ix + 0.5) * s

    for j in range(_NG):
        ax = gt_ref[b, j, 0, 0]
        ay = gt_ref[b, j, 0, 1]
        bx = gt_ref[b, j, 1, 0]
        by = gt_ref[b, j, 1, 1]
        cx = gt_ref[b, j, 2, 0]
        cy = gt_ref[b, j, 2, 1]

        def sign(x1, y1, x2, y2, x3, y3):
            return (x1 - x3) * (y2 - y3) - (x2 - x3) * (y1 - y3)

        d1 = sign(xx, yy, ax, ay, bx, by)
        d2 = sign(xx, yy, bx, by, cx, cy)
        d3 = sign(xx, yy, cx, cy, ax, ay)
        has_neg = (d1 < 0) | (d2 < 0) | (d3 < 0)
        has_pos = (d1 > 0) | (d2 > 0) | (d3 > 0)
        inside = ~(has_neg & has_pos)
        dist = jnp.minimum(_seg_dist(xx, yy, ax, ay, bx, by),
                           jnp.minimum(_seg_dist(xx, yy, bx, by, cx, cy),
                                       _seg_dist(xx, yy, cx, cy, ax, ay)))
        mask = inside | (dist <= _ETA_PX)
        dist_bits = jax.lax.bitcast_convert_type(dist, jnp.int32)
        key_ref[j] = jnp.where(mask, dist_bits, jnp.int32(_MAXI))

        g0x = (ax - xx) / s
        g0y = (ay - yy) / s
        p0 = (reg_ref[0, 0] - g0x) ** 2 + (reg_ref[0, 1] - g0y) ** 2

        g1x = (bx - xx) / s
        g1y = (by - yy) / s
        g2x = (cx - xx) / s
        g2y = (cy - yy) / s
        p1x, p1y = reg_ref[0, 2], reg_ref[0, 3]
        p2x, p2y = reg_ref[0, 4], reg_ref[0, 5]

        def pdist(px_, py_, gx_, gy_):
            return jnp.sqrt((px_ - gx_) ** 2 + (py_ - gy_) ** 2 + 1e-12)

        d11 = pdist(p1x, p1y, g1x, g1y)
        d12 = pdist(p1x, p1y, g2x, g2y)
        d21 = pdist(p2x, p2y, g1x, g1y)
        d22 = pdist(p2x, p2y, g2x, g2y)
        cd = (jnp.minimum(d11, d12) + jnp.minimum(d21, d22)
              + jnp.minimum(d11, d21) + jnp.minimum(d12, d22))
        loss_ref[j] = _LAMBDA_P0 * p0 + _LAMBDA_CD * cd

    key3 = key_ref[...]
    mask3 = key3 != _MAXI

    def _cnt(x):
        return jnp.sum(jnp.sum(x.astype(jnp.int32), axis=2, keepdims=True),
                       axis=1, keepdims=True)

    npix3 = _cnt(mask3)

    def bs_body(_, carry):
        lo, hi = carry
        mid = lo + (hi - lo) // 2
        cnt = _cnt(key_ref[...] <= mid)
        take = cnt >= _K_POS_CAP
        return (jnp.where(take, lo, mid + 1), jnp.where(take, mid, hi))

    lo, _hi = jax.lax.fori_loop(
        0, 31, bs_body,
        (jnp.zeros((_NG, 1, 1), jnp.int32),
         jnp.full((_NG, 1, 1), _FINF_BITS, jnp.int32)))

    mf3 = ((key3 <= lo) | ((npix3 <= _K_POS_CAP) & mask3)).astype(jnp.float32)
    obj_t = jnp.max(mf3, axis=0)
    mf_sum = jnp.sum(mf3, axis=0)
    reg_l = jnp.sum(mf3 * loss_ref[...])
    pos = jnp.sum(jnp.minimum(npix3, _K_POS_CAP)).astype(jnp.float32)

    x = obj_ref[0, 0]
    obj_l = jnp.sum(-(_POS_W * obj_t * _log_sigmoid(x)
                      + (1.0 - obj_t) * _log_sigmoid(-x)))
    cls_l = jnp.sum(mf_sum * (-_log_sigmoid(cls_ref[0, 0])))
    neg = jnp.float32(_H * _W) - jnp.sum((obj_t > 0.5).astype(jnp.float32))

    acc_ref[0] = acc_ref[0] + reg_l
    acc_ref[1] = acc_ref[1] + obj_l
    acc_ref[2] = acc_ref[2] + cls_l
    acc_ref[3] = acc_ref[3] + pos
    acc_ref[4] = acc_ref[4] + neg

    @pl.when(b == _B - 1)
    def _finalize():
        pos_eps = jnp.maximum(acc_ref[3], 1.0)
        neg_eps = jnp.maximum(acc_ref[4], 1.0)
        out_ref[0] = (acc_ref[0] / pos_eps
                      + acc_ref[1] / (pos_eps + neg_eps)
                      + acc_ref[2] / pos_eps)


def kernel(pred_reg, pred_obj, pred_cls, gt_points, stride):
    s = jnp.asarray(stride, jnp.float32).reshape(1)
    out = pl.pallas_call(
        _loss_kernel,
        grid=(_B,),
        in_specs=[
            pl.BlockSpec(memory_space=pltpu.SMEM),
            pl.BlockSpec(memory_space=pltpu.SMEM),
            pl.BlockSpec((1, 6, _H, _W), lambda b: (b, 0, 0, 0)),
            pl.BlockSpec((1, 1, _H, _W), lambda b: (b, 0, 0, 0)),
            pl.BlockSpec((1, 1, _H, _W), lambda b: (b, 0, 0, 0)),
        ],
        out_specs=pl.BlockSpec(memory_space=pltpu.SMEM),
        out_shape=jax.ShapeDtypeStruct((1,), jnp.float32),
        scratch_shapes=[
            pltpu.SMEM((5,), jnp.float32),
            pltpu.VMEM((_NG, _H, _W), jnp.int32),
            pltpu.VMEM((_NG, _H, _W), jnp.float32),
        ],
    )(gt_points, s, pred_reg, pred_obj, pred_cls)
    return out[0]


# speedup vs baseline: 71.2955x; 1.6666x over previous
import jax
import jax.numpy as jnp
import numpy as np
from jax.experimental import pallas as pl
from jax.experimental.pallas import tpu as pltpu

_B = 8
_NG = 8
_H = 128
_W = 128
_ETA_PX = 3.0
_POS_W = 1.2
_LAMBDA_CD = 1.0
_K_POS_CAP = 96
_LAMBDA_P0 = 1.0
_MAXI = 0x7FFFFFFF
_LO_BITS = int(np.float32(1e-13).view(np.int32))
_HI_BITS = int(np.float32(1048576.0).view(np.int32))
_BS_ITERS = int(np.ceil(np.log2(float(_HI_BITS - _LO_BITS))))


def _log_sigmoid(x):
    return jnp.minimum(x, 0.0) - jnp.log1p(jnp.exp(-jnp.abs(x)))


def _seg_dist2(px, py, x1, y1, x2, y2):
    vx, vy = x2 - x1, y2 - y1
    wx, wy = px - x1, py - y1
    vv = vx * vx + vy * vy + 1e-09
    t = jnp.clip((wx * vx + wy * vy) * (1.0 / vv), 0.0, 1.0)
    dx = px - (x1 + t * vx)
    dy = py - (y1 + t * vy)
    return dx * dx + dy * dy + 1e-12


def _loss_kernel(gt_ref, s_ref, reg_ref, obj_ref, cls_ref, out_ref,
                 acc_ref, key_ref, loss_ref):
    b = pl.program_id(0)
    s = s_ref[0]

    @pl.when(b == 0)
    def _init():
        for i in range(5):
            acc_ref[i] = 0.0

    iy = jax.lax.broadcasted_iota(jnp.int32, (_H, _W), 0).astype(jnp.float32)
    ix = jax.lax.broadcasted_iota(jnp.int32, (_H, _W), 1).astype(jnp.float32)
    yy = (iy + 0.5) * s
    xx = (ix + 0.5) * s

    for j in range(_NG):
        ax = gt_ref[b, j, 0, 0]
        ay = gt_ref[b, j, 0, 1]
        bx = gt_ref[b, j, 1, 0]
        by = gt_ref[b, j, 1, 1]
        cx = gt_ref[b, j, 2, 0]
        cy = gt_ref[b, j, 2, 1]

        def sign(x1, y1, x2, y2, x3, y3):
            return (x1 - x3) * (y2 - y3) - (x2 - x3) * (y1 - y3)

        d1 = sign(xx, yy, ax, ay, bx, by)
        d2 = sign(xx, yy, bx, by, cx, cy)
        d3 = sign(xx, yy, cx, cy, ax, ay)
        has_neg = (d1 < 0) | (d2 < 0) | (d3 < 0)
        has_pos = (d1 > 0) | (d2 > 0) | (d3 > 0)
        inside = ~(has_neg & has_pos)
        d2 = jnp.minimum(_seg_dist2(xx, yy, ax, ay, bx, by),
                         jnp.minimum(_seg_dist2(xx, yy, bx, by, cx, cy),
                                     _seg_dist2(xx, yy, cx, cy, ax, ay)))
        mask = inside | (d2 <= _ETA_PX * _ETA_PX)
        d2_bits = jax.lax.bitcast_convert_type(d2, jnp.int32)
        key_ref[j] = jnp.where(mask, d2_bits, jnp.int32(_MAXI))

        g0x = (ax - xx) / s
        g0y = (ay - yy) / s
        p0 = (reg_ref[0, 0] - g0x) ** 2 + (reg_ref[0, 1] - g0y) ** 2

        g1x = (bx - xx) / s
        g1y = (by - yy) / s
        g2x = (cx - xx) / s
        g2y = (cy - yy) / s
        p1x, p1y = reg_ref[0, 2], reg_ref[0, 3]
        p2x, p2y = reg_ref[0, 4], reg_ref[0, 5]

        def pdist(px_, py_, gx_, gy_):
            return jnp.sqrt((px_ - gx_) ** 2 + (py_ - gy_) ** 2 + 1e-12)

        d11 = pdist(p1x, p1y, g1x, g1y)
        d12 = pdist(p1x, p1y, g2x, g2y)
        d21 = pdist(p2x, p2y, g1x, g1y)
        d22 = pdist(p2x, p2y, g2x, g2y)
        cd = (jnp.minimum(d11, d12) + jnp.minimum(d21, d22)
              + jnp.minimum(d11, d21) + jnp.minimum(d12, d22))
        loss_ref[j] = _LAMBDA_P0 * p0 + _LAMBDA_CD * cd

    key3 = key_ref[...]
    mask3 = key3 != _MAXI

    def _cnt(x):
        return jnp.sum(jnp.sum(x.astype(jnp.int32), axis=1, keepdims=True),
                       axis=2, keepdims=True)

    npix3 = _cnt(mask3)

    def bs_body(_, carry):
        lo, hi = carry
        mid = lo + (hi - lo) // 2
        cnt = _cnt(key_ref[...] <= mid)
        take = cnt >= _K_POS_CAP
        return (jnp.where(take, lo, mid + 1), jnp.where(take, mid, hi))

    lo, _hi = jax.lax.fori_loop(
        0, _BS_ITERS, bs_body,
        (jnp.full((_NG, 1, 1), _LO_BITS, jnp.int32),
         jnp.full((_NG, 1, 1), _HI_BITS, jnp.int32)))

    mf3 = ((key3 <= lo) | ((npix3 <= _K_POS_CAP) & mask3)).astype(jnp.float32)
    obj_t = jnp.max(mf3, axis=0)
    mf_sum = jnp.sum(mf3, axis=0)
    reg_l = jnp.sum(mf3 * loss_ref[...])
    pos = jnp.sum(jnp.minimum(npix3, _K_POS_CAP)).astype(jnp.float32)

    x = obj_ref[0, 0]
    obj_l = jnp.sum(-(_POS_W * obj_t * _log_sigmoid(x)
                      + (1.0 - obj_t) * _log_sigmoid(-x)))
    cls_l = jnp.sum(mf_sum * (-_log_sigmoid(cls_ref[0, 0])))
    neg = jnp.float32(_H * _W) - jnp.sum((obj_t > 0.5).astype(jnp.float32))

    acc_ref[0] = acc_ref[0] + reg_l
    acc_ref[1] = acc_ref[1] + obj_l
    acc_ref[2] = acc_ref[2] + cls_l
    acc_ref[3] = acc_ref[3] + pos
    acc_ref[4] = acc_ref[4] + neg

    @pl.when(b == _B - 1)
    def _finalize():
        pos_eps = jnp.maximum(acc_ref[3], 1.0)
        neg_eps = jnp.maximum(acc_ref[4], 1.0)
        out_ref[0] = (acc_ref[0] / pos_eps
                      + acc_ref[1] / (pos_eps + neg_eps)
                      + acc_ref[2] / pos_eps)


def kernel(pred_reg, pred_obj, pred_cls, gt_points, stride):
    s = jnp.asarray(stride, jnp.float32).reshape(1)
    out = pl.pallas_call(
        _loss_kernel,
        grid=(_B,),
        in_specs=[
            pl.BlockSpec(memory_space=pltpu.SMEM),
            pl.BlockSpec(memory_space=pltpu.SMEM),
            pl.BlockSpec((1, 6, _H, _W), lambda b: (b, 0, 0, 0)),
            pl.BlockSpec((1, 1, _H, _W), lambda b: (b, 0, 0, 0)),
            pl.BlockSpec((1, 1, _H, _W), lambda b: (b, 0, 0, 0)),
        ],
        out_specs=pl.BlockSpec(memory_space=pltpu.SMEM),
        out_shape=jax.ShapeDtypeStruct((1,), jnp.float32),
        scratch_shapes=[
            pltpu.SMEM((5,), jnp.float32),
            pltpu.VMEM((_NG, _H, _W), jnp.int32),
            pltpu.VMEM((_NG, _H, _W), jnp.float32),
        ],
    )(gt_points, s, pred_reg, pred_obj, pred_cls)
    return out[0]


# speedup vs baseline: 76.8962x; 1.0786x over previous
import jax
import jax.numpy as jnp
import numpy as np
from jax.experimental import pallas as pl
from jax.experimental.pallas import tpu as pltpu

_B = 8
_NG = 8
_H = 128
_W = 128
_ETA_PX = 3.0
_POS_W = 1.2
_LAMBDA_CD = 1.0
_K_POS_CAP = 96
_LAMBDA_P0 = 1.0
_MAXI = 0x7FFFFFFF
_LO_BITS = int(np.float32(1e-13).view(np.int32))
_HI_BITS = int(np.float32(1048576.0).view(np.int32))
_BS_ITERS = int(np.ceil(np.log2(float(_HI_BITS - _LO_BITS))))


def _log_sigmoid(x):
    return jnp.minimum(x, 0.0) - jnp.log1p(jnp.exp(-jnp.abs(x)))


def _seg_dist2(px, py, x1, y1, x2, y2):
    vx, vy = x2 - x1, y2 - y1
    wx, wy = px - x1, py - y1
    vv = vx * vx + vy * vy + 1e-09
    t = jnp.clip((wx * vx + wy * vy) * (1.0 / vv), 0.0, 1.0)
    dx = px - (x1 + t * vx)
    dy = py - (y1 + t * vy)
    return dx * dx + dy * dy + 1e-12


def _loss_kernel(gt_ref, s_ref, reg_ref, obj_ref, cls_ref, out_ref,
                 acc_ref, key_ref, loss_ref):
    b = pl.program_id(0)
    s = s_ref[0]

    @pl.when(b == 0)
    def _init():
        for i in range(5):
            acc_ref[i] = 0.0

    iy = jax.lax.broadcasted_iota(jnp.int32, (_H, _W), 0).astype(jnp.float32)
    ix = jax.lax.broadcasted_iota(jnp.int32, (_H, _W), 1).astype(jnp.float32)
    yy = (iy + 0.5) * s
    xx = (ix + 0.5) * s

    for j in range(_NG):
        ax = gt_ref[b, j, 0, 0]
        ay = gt_ref[b, j, 0, 1]
        bx = gt_ref[b, j, 1, 0]
        by = gt_ref[b, j, 1, 1]
        cx = gt_ref[b, j, 2, 0]
        cy = gt_ref[b, j, 2, 1]

        def sign(x1, y1, x2, y2, x3, y3):
            return (x1 - x3) * (y2 - y3) - (x2 - x3) * (y1 - y3)

        d1 = sign(xx, yy, ax, ay, bx, by)
        d2 = sign(xx, yy, bx, by, cx, cy)
        d3 = sign(xx, yy, cx, cy, ax, ay)
        has_neg = (d1 < 0) | (d2 < 0) | (d3 < 0)
        has_pos = (d1 > 0) | (d2 > 0) | (d3 > 0)
        inside = ~(has_neg & has_pos)
        d2 = jnp.minimum(_seg_dist2(xx, yy, ax, ay, bx, by),
                         jnp.minimum(_seg_dist2(xx, yy, bx, by, cx, cy),
                                     _seg_dist2(xx, yy, cx, cy, ax, ay)))
        mask = inside | (d2 <= _ETA_PX * _ETA_PX)
        d2_bits = jax.lax.bitcast_convert_type(d2, jnp.int32)
        key_ref[j] = jnp.where(mask, d2_bits, jnp.int32(_MAXI))

        g0x = (ax - xx) / s
        g0y = (ay - yy) / s
        p0 = (reg_ref[0, 0] - g0x) ** 2 + (reg_ref[0, 1] - g0y) ** 2

        g1x = (bx - xx) / s
        g1y = (by - yy) / s
        g2x = (cx - xx) / s
        g2y = (cy - yy) / s
        p1x, p1y = reg_ref[0, 2], reg_ref[0, 3]
        p2x, p2y = reg_ref[0, 4], reg_ref[0, 5]

        def pdist(px_, py_, gx_, gy_):
            return jnp.sqrt((px_ - gx_) ** 2 + (py_ - gy_) ** 2 + 1e-12)

        d11 = pdist(p1x, p1y, g1x, g1y)
        d12 = pdist(p1x, p1y, g2x, g2y)
        d21 = pdist(p2x, p2y, g1x, g1y)
        d22 = pdist(p2x, p2y, g2x, g2y)
        cd = (jnp.minimum(d11, d12) + jnp.minimum(d21, d22)
              + jnp.minimum(d11, d21) + jnp.minimum(d12, d22))
        loss_ref[j] = _LAMBDA_P0 * p0 + _LAMBDA_CD * cd

    key3 = key_ref[...]
    mask3 = key3 != _MAXI

    def _cnt(x):
        part = jnp.sum(x.astype(jnp.int32), axis=1)
        return jnp.sum(part, axis=1, keepdims=True)

    npix2 = _cnt(mask3)

    def bs_body(_, carry):
        lo, hi = carry
        mid = lo + (hi - lo) // 2
        cnt = _cnt(key_ref[...] <= mid.reshape(_NG, 1, 1))
        take = cnt >= _K_POS_CAP
        return (jnp.where(take, lo, mid + 1), jnp.where(take, mid, hi))

    lo2, _hi = jax.lax.fori_loop(
        0, _BS_ITERS, bs_body,
        (jnp.full((_NG, 1), _LO_BITS, jnp.int32),
         jnp.full((_NG, 1), _HI_BITS, jnp.int32)))
    lo = lo2.reshape(_NG, 1, 1)
    npix3 = npix2.reshape(_NG, 1, 1)

    mf3 = ((key3 <= lo) | ((npix3 <= _K_POS_CAP) & mask3)).astype(jnp.float32)
    obj_t = jnp.max(mf3, axis=0)
    mf_sum = jnp.sum(mf3, axis=0)
    reg_l = jnp.sum(mf3 * loss_ref[...])
    pos = jnp.sum(jnp.minimum(npix3, _K_POS_CAP)).astype(jnp.float32)

    x = obj_ref[0, 0]
    obj_l = jnp.sum(-(_POS_W * obj_t * _log_sigmoid(x)
                      + (1.0 - obj_t) * _log_sigmoid(-x)))
    cls_l = jnp.sum(mf_sum * (-_log_sigmoid(cls_ref[0, 0])))
    neg = jnp.float32(_H * _W) - jnp.sum((obj_t > 0.5).astype(jnp.float32))

    acc_ref[0] = acc_ref[0] + reg_l
    acc_ref[1] = acc_ref[1] + obj_l
    acc_ref[2] = acc_ref[2] + cls_l
    acc_ref[3] = acc_ref[3] + pos
    acc_ref[4] = acc_ref[4] + neg

    @pl.when(b == _B - 1)
    def _finalize():
        pos_eps = jnp.maximum(acc_ref[3], 1.0)
        neg_eps = jnp.maximum(acc_ref[4], 1.0)
        out_ref[0] = (acc_ref[0] / pos_eps
                      + acc_ref[1] / (pos_eps + neg_eps)
                      + acc_ref[2] / pos_eps)


def kernel(pred_reg, pred_obj, pred_cls, gt_points, stride):
    s = jnp.asarray(stride, jnp.float32).reshape(1)
    out = pl.pallas_call(
        _loss_kernel,
        grid=(_B,),
        in_specs=[
            pl.BlockSpec(memory_space=pltpu.SMEM),
            pl.BlockSpec(memory_space=pltpu.SMEM),
            pl.BlockSpec((1, 6, _H, _W), lambda b: (b, 0, 0, 0)),
            pl.BlockSpec((1, 1, _H, _W), lambda b: (b, 0, 0, 0)),
            pl.BlockSpec((1, 1, _H, _W), lambda b: (b, 0, 0, 0)),
        ],
        out_specs=pl.BlockSpec(memory_space=pltpu.SMEM),
        out_shape=jax.ShapeDtypeStruct((1,), jnp.float32),
        scratch_shapes=[
            pltpu.SMEM((5,), jnp.float32),
            pltpu.VMEM((_NG, _H, _W), jnp.int32),
            pltpu.VMEM((_NG, _H, _W), jnp.float32),
        ],
    )(gt_points, s, pred_reg, pred_obj, pred_cls)
    return out[0]
